# Initial kernel scaffold; baseline (speedup 1.0000x reference)
import jax
import jax.numpy as jnp
from jax.experimental import pallas as pl


def kernel(M, lengths, edge_ind, W):
    raise NotImplementedError("write your pallas kernel here")


# speedup vs baseline: 2.6869x; 2.6869x over previous
import functools

import jax
import jax.numpy as jnp
from jax.experimental import pallas as pl

S, B, D = 512, 32, 1024
L = 512


def _mea_kernel(e0_ref, e1_ref, m_ref, w_ref, out_ref):
    Mb = m_ref[...]
    W = w_ref[...]
    scale_t = jax.lax.dot_general(
        W, Mb,
        dimension_numbers=(((1,), (1,)), ((), ())),
        preferred_element_type=jnp.float32,
        precision=jax.lax.Precision.HIGHEST,
    )
    mx = jnp.max(scale_t, axis=1, keepdims=True)
    ex = jnp.exp(scale_t - mx)
    z = jnp.sum(ex, axis=1, keepdims=True)
    alpha_t = ex / z

    e0 = jnp.minimum(e0_ref[0, 0, :], L - 1)
    e1 = jnp.minimum(e1_ref[0, 0, :], S - 1)
    E = e0.shape[0]
    rows = jax.lax.broadcasted_iota(jnp.int32, (L, E), 0)
    cols = jax.lax.broadcasted_iota(jnp.int32, (E, S), 1)
    P = (rows == e0[None, :]).astype(jnp.bfloat16)
    Q = (cols == e1[:, None]).astype(jnp.bfloat16)
    C = jax.lax.dot_general(
        P, Q,
        dimension_numbers=(((1,), (0,)), ((), ())),
        preferred_element_type=jnp.float32,
    )
    hit = C > 0.0

    masked = jnp.where(hit, alpha_t, 0.0)
    sum_e = jnp.sum(masked, axis=1, keepdims=True)
    row_total = jnp.sum(alpha_t, axis=1, keepdims=True)
    denom = sum_e + 1e-10 * (row_total - sum_e) + 1e-10
    out_ref[0, :, :] = jnp.where(hit, alpha_t / denom, 0.0)


@jax.jit
def kernel(M, lengths, edge_ind, W):
    del lengths
    e0 = edge_ind[:, :, 0].astype(jnp.int32).reshape(B, 1, -1)
    e1 = edge_ind[:, :, 1].astype(jnp.int32).reshape(B, 1, -1)
    E = e0.shape[-1]
    M2 = M.reshape(S, B * D)
    grid = (B,)
    return pl.pallas_call(
        _mea_kernel,
        grid=grid,
        in_specs=[
            pl.BlockSpec((1, 1, E), lambda b: (b, 0, 0)),
            pl.BlockSpec((1, 1, E), lambda b: (b, 0, 0)),
            pl.BlockSpec((S, D), lambda b: (0, b)),
            pl.BlockSpec((L, D), lambda b: (0, 0)),
        ],
        out_specs=pl.BlockSpec((1, L, S), lambda b: (b, 0, 0)),
        out_shape=jax.ShapeDtypeStruct((B, L, S), jnp.float32),
    )(e0, e1, M2, W)


# speedup vs baseline: 4.3656x; 1.6248x over previous
import functools

import jax
import jax.numpy as jnp
from jax.experimental import pallas as pl

S, B, D = 512, 32, 1024
L = 512


def _mea_kernel(e0_ref, e1_ref, m_ref, w_ref, out_ref):
    Mb = m_ref[...]
    W = w_ref[...]
    scale_t = jax.lax.dot_general(
        W, Mb,
        dimension_numbers=(((1,), (1,)), ((), ())),
        preferred_element_type=jnp.float32,
    )
    mx = jnp.max(scale_t, axis=1, keepdims=True)
    ex = jnp.exp(scale_t - mx)
    z = jnp.sum(ex, axis=1, keepdims=True)
    alpha_t = ex / z

    e0 = jnp.minimum(e0_ref[0, 0, :], L - 1)
    e1 = jnp.minimum(e1_ref[0, 0, :], S - 1)
    E = e0.shape[0]
    rows = jax.lax.broadcasted_iota(jnp.int32, (L, E), 0)
    cols = jax.lax.broadcasted_iota(jnp.int32, (E, S), 1)
    P = (rows == e0[None, :]).astype(jnp.bfloat16)
    Q = (cols == e1[:, None]).astype(jnp.bfloat16)
    C = jax.lax.dot_general(
        P, Q,
        dimension_numbers=(((1,), (0,)), ((), ())),
        preferred_element_type=jnp.float32,
    )
    hit = C > 0.0

    masked = jnp.where(hit, alpha_t, 0.0)
    sum_e = jnp.sum(masked, axis=1, keepdims=True)
    row_total = jnp.sum(alpha_t, axis=1, keepdims=True)
    denom = sum_e + 1e-10 * (row_total - sum_e) + 1e-10
    out_ref[0, :, :] = jnp.where(hit, alpha_t / denom, 0.0)


@jax.jit
def kernel(M, lengths, edge_ind, W):
    del lengths
    e0 = edge_ind[:, :, 0].astype(jnp.int32).reshape(B, 1, -1)
    e1 = edge_ind[:, :, 1].astype(jnp.int32).reshape(B, 1, -1)
    E = e0.shape[-1]
    M2 = M.reshape(S, B * D)
    grid = (B,)
    return pl.pallas_call(
        _mea_kernel,
        grid=grid,
        in_specs=[
            pl.BlockSpec((1, 1, E), lambda b: (b, 0, 0)),
            pl.BlockSpec((1, 1, E), lambda b: (b, 0, 0)),
            pl.BlockSpec((S, D), lambda b: (0, b)),
            pl.BlockSpec((L, D), lambda b: (0, 0)),
        ],
        out_specs=pl.BlockSpec((1, L, S), lambda b: (b, 0, 0)),
        out_shape=jax.ShapeDtypeStruct((B, L, S), jnp.float32),
    )(e0, e1, M2, W)
